# baseline (speedup 1.0000x reference)
import functools

import jax
import jax.numpy as jnp
from jax import lax
from jax.experimental import pallas as pl
from jax.experimental.pallas import tpu as pltpu
from jax.experimental.pallas import tpu_sc as plsc

VOCAB_ = 100000
DIM_ = 64
B_ = 4096

_NC = 2
_NS = 16
_NW = _NC * _NS
_BPW = B_ // _NW


def _sc_gather_body(focal, context, hfe, hce, sfe, sce, bias_t,
                    hf_o, hc_o, sf_o, sc_o, bf_o, bc_o,
                    idxf_v, idxc_v, r0, r1, r2, r3, b0, b1, sem):
    wid = lax.axis_index("s") * _NC + lax.axis_index("c")
    base = wid * _BPW
    pltpu.sync_copy(focal.at[pl.ds(base, _BPW)], idxf_v)
    pltpu.sync_copy(context.at[pl.ds(base, _BPW)], idxc_v)
    cps = [
        pltpu.async_copy(hfe.at[idxf_v], r0, sem),
        pltpu.async_copy(hce.at[idxc_v], r1, sem),
        pltpu.async_copy(sfe.at[idxf_v], r2, sem),
        pltpu.async_copy(sce.at[idxc_v], r3, sem),
        pltpu.async_copy(bias_t.at[idxf_v], b0, sem),
        pltpu.async_copy(bias_t.at[idxc_v], b1, sem),
    ]
    for cp in cps:
        cp.wait()
    s = pl.ds(base, _BPW)
    pltpu.sync_copy(r0, hf_o.at[s])
    pltpu.sync_copy(r1, hc_o.at[s])
    pltpu.sync_copy(r2, sf_o.at[s])
    pltpu.sync_copy(r3, sc_o.at[s])
    pltpu.sync_copy(b0, bf_o.at[s])
    pltpu.sync_copy(b1, bc_o.at[s])


_f32 = jnp.float32


@functools.lru_cache(maxsize=1)
def _make_sc_gather():
    return functools.partial(
        pl.kernel,
        mesh=plsc.VectorSubcoreMesh(core_axis_name="c", subcore_axis_name="s"),
        compiler_params=pltpu.CompilerParams(use_tc_tiling_on_sc=False),
        out_type=[jax.ShapeDtypeStruct((B_, DIM_), _f32)] * 4
               + [jax.ShapeDtypeStruct((B_, 8), _f32)] * 2,
        scratch_types=[
            pltpu.VMEM((_BPW,), jnp.int32),
            pltpu.VMEM((_BPW,), jnp.int32),
            pltpu.VMEM((_BPW, DIM_), _f32),
            pltpu.VMEM((_BPW, DIM_), _f32),
            pltpu.VMEM((_BPW, DIM_), _f32),
            pltpu.VMEM((_BPW, DIM_), _f32),
            pltpu.VMEM((_BPW, 8), _f32),
            pltpu.VMEM((_BPW, 8), _f32),
            pltpu.SemaphoreType.DMA,
        ],
    )(_sc_gather_body)


_SK = 0.7071067811865476


def _stereo_nm(xr, yr, k):
    x = xr / jnp.sqrt(jnp.sum(xr * xr, axis=1, keepdims=True))
    y = yr / jnp.sqrt(jnp.sum(yr * yr, axis=1, keepdims=True))
    x2 = jnp.sum(x * x, axis=-1, keepdims=True)
    y2 = jnp.sum(y * y, axis=-1, keepdims=True)
    xy = jnp.sum(x * y, axis=-1, keepdims=True)
    num = (1.0 + k * x2) * y - (1.0 + 2.0 * k * xy - k * y2) * x
    den = 1.0 + 2.0 * k * xy + (k * k) * x2 * y2
    m = num / jnp.maximum(den, 1e-15)
    return jnp.sqrt(jnp.sum(m * m, axis=-1, keepdims=True))


def _atan_pos(z):
    big = z > 1.0
    w = jnp.where(big, 1.0 / jnp.maximum(z, 1e-30), z)
    w = w / (1.0 + jnp.sqrt(1.0 + w * w))
    w = w / (1.0 + jnp.sqrt(1.0 + w * w))
    w2 = w * w
    p = w * (1.0 + w2 * (-1.0 / 3.0 + w2 * (0.2 + w2 * (-1.0 / 7.0
                                                        + w2 / 9.0))))
    a = 4.0 * p
    return jnp.where(big, 1.5707963267948966 - a, a)


def _rowmath_body(hf, hc, sf, sc_, bias_f, bias_c, logc,
                  th_o, ts_o, le_o, dh_o, ds_o):
    lg = logc[...]
    bf = bias_f[...]
    bc = bias_c[...]
    th_o[...] = bf[:, 2:3] + bc[:, 3:4] - lg
    ts_o[...] = bf[:, 4:5] + bc[:, 5:6] - lg
    te = bf[:, 0:1] + bc[:, 1:2] - lg
    le_o[...] = (te - 1.0) ** 2

    nmh = _stereo_nm(hf[...], hc[...], -0.5)
    t = jnp.clip(_SK * nmh, -1.0 + 1e-5, 1.0 - 1e-5)
    dist = jnp.log((1.0 + t) / (1.0 - t)) / _SK
    e = jnp.exp(dist)
    ch = 0.5 * (e + 1.0 / e)
    vh = ch * ch
    dh_o[...] = vh / jnp.sqrt(jnp.sum(vh * vh))

    nms = _stereo_nm(sf[...], sc_[...], 0.5)
    dsts = 2.0 * _atan_pos(_SK * nms) / _SK
    vs = 0.5 * dsts * dsts
    ds_o[...] = vs / jnp.sqrt(jnp.sum(vs * vs))


_rowmath = pl.pallas_call(
    _rowmath_body,
    out_shape=[jax.ShapeDtypeStruct((B_, 1), _f32)] * 5,
)


_TI = 256


def _outer_body(th, ts, le, dh, ds, out):
    out[...] = (0.5 * le[...]
                + 0.25 * (th[...] - dh[...]) ** 2
                + 0.25 * (ts[...] - ds[...]) ** 2)


_outer = pl.pallas_call(
    _outer_body,
    grid=(B_ // _TI,),
    in_specs=[
        pl.BlockSpec((_TI, 1), lambda i: (i, 0)),
        pl.BlockSpec((_TI, 1), lambda i: (i, 0)),
        pl.BlockSpec((_TI, 1), lambda i: (i, 0)),
        pl.BlockSpec((1, B_), lambda i: (0, 0)),
        pl.BlockSpec((1, B_), lambda i: (0, 0)),
    ],
    out_specs=pl.BlockSpec((_TI, B_), lambda i: (i, 0)),
    out_shape=jax.ShapeDtypeStruct((B_, B_), _f32),
)


def kernel(focal_input, context_input, log_coocurrence_count,
           euc_fe, euc_ce, euc_fb, euc_cb,
           hyp_fe, hyp_ce, hyp_fb, hyp_cb,
           sph_fe, sph_ce, sph_fb, sph_cb):
    f = focal_input.astype(jnp.int32)
    c = context_input.astype(jnp.int32)
    bias_t = jnp.concatenate(
        [euc_fb, euc_cb, hyp_fb, hyp_cb, sph_fb, sph_cb, euc_fb, euc_cb],
        axis=1)
    (hf, hc, sf, sc_, bias_f, bias_c) = _make_sc_gather()(
        f, c, hyp_fe, hyp_ce, sph_fe, sph_ce, bias_t)
    th, ts, le, dh, ds = _rowmath(hf, hc, sf, sc_, bias_f, bias_c,
                                  log_coocurrence_count)
    return _outer(th, ts, le, dh.reshape(1, B_), ds.reshape(1, B_))


# speedup vs baseline: 1.0074x; 1.0074x over previous
import functools

import jax
import jax.numpy as jnp
from jax import lax
from jax.experimental import pallas as pl
from jax.experimental.pallas import tpu as pltpu
from jax.experimental.pallas import tpu_sc as plsc

VOCAB_ = 100000
DIM_ = 64
B_ = 4096

_NC = 2
_NS = 16
_NW = _NC * _NS
_BPW = B_ // _NW


def _sc_gather_body(focal, context, hfe, hce, sfe, sce, bias_t,
                    hf_o, hc_o, sf_o, sc_o, bf_o, bc_o,
                    idxf_v, idxc_v, r0, r1, r2, r3, b0, b1, sem):
    wid = lax.axis_index("s") * _NC + lax.axis_index("c")
    base = wid * _BPW
    pltpu.sync_copy(focal.at[pl.ds(base, _BPW)], idxf_v)
    pltpu.sync_copy(context.at[pl.ds(base, _BPW)], idxc_v)
    cps = [
        pltpu.async_copy(hfe.at[idxf_v], r0, sem),
        pltpu.async_copy(hce.at[idxc_v], r1, sem),
        pltpu.async_copy(sfe.at[idxf_v], r2, sem),
        pltpu.async_copy(sce.at[idxc_v], r3, sem),
        pltpu.async_copy(bias_t.at[idxf_v], b0, sem),
        pltpu.async_copy(bias_t.at[idxc_v], b1, sem),
    ]
    for cp in cps:
        cp.wait()
    s = pl.ds(base, _BPW)
    pltpu.sync_copy(r0, hf_o.at[s])
    pltpu.sync_copy(r1, hc_o.at[s])
    pltpu.sync_copy(r2, sf_o.at[s])
    pltpu.sync_copy(r3, sc_o.at[s])
    pltpu.sync_copy(b0, bf_o.at[s])
    pltpu.sync_copy(b1, bc_o.at[s])


_f32 = jnp.float32


@functools.lru_cache(maxsize=1)
def _make_sc_gather():
    return functools.partial(
        pl.kernel,
        mesh=plsc.VectorSubcoreMesh(core_axis_name="c", subcore_axis_name="s"),
        compiler_params=pltpu.CompilerParams(use_tc_tiling_on_sc=False),
        out_type=[jax.ShapeDtypeStruct((B_, DIM_), _f32)] * 4
               + [jax.ShapeDtypeStruct((B_, 8), _f32)] * 2,
        scratch_types=[
            pltpu.VMEM((_BPW,), jnp.int32),
            pltpu.VMEM((_BPW,), jnp.int32),
            pltpu.VMEM((_BPW, DIM_), _f32),
            pltpu.VMEM((_BPW, DIM_), _f32),
            pltpu.VMEM((_BPW, DIM_), _f32),
            pltpu.VMEM((_BPW, DIM_), _f32),
            pltpu.VMEM((_BPW, 8), _f32),
            pltpu.VMEM((_BPW, 8), _f32),
            pltpu.SemaphoreType.DMA,
        ],
    )(_sc_gather_body)


_SK = 0.7071067811865476


def _stereo_nm(xr, yr, k):
    x = xr / jnp.sqrt(jnp.sum(xr * xr, axis=1, keepdims=True))
    y = yr / jnp.sqrt(jnp.sum(yr * yr, axis=1, keepdims=True))
    x2 = jnp.sum(x * x, axis=-1, keepdims=True)
    y2 = jnp.sum(y * y, axis=-1, keepdims=True)
    xy = jnp.sum(x * y, axis=-1, keepdims=True)
    num = (1.0 + k * x2) * y - (1.0 + 2.0 * k * xy - k * y2) * x
    den = 1.0 + 2.0 * k * xy + (k * k) * x2 * y2
    m = num / jnp.maximum(den, 1e-15)
    return jnp.sqrt(jnp.sum(m * m, axis=-1, keepdims=True))


def _atan_pos(z):
    big = z > 1.0
    w = jnp.where(big, 1.0 / jnp.maximum(z, 1e-30), z)
    w = w / (1.0 + jnp.sqrt(1.0 + w * w))
    w = w / (1.0 + jnp.sqrt(1.0 + w * w))
    w2 = w * w
    p = w * (1.0 + w2 * (-1.0 / 3.0 + w2 * (0.2 + w2 * (-1.0 / 7.0
                                                        + w2 / 9.0))))
    a = 4.0 * p
    return jnp.where(big, 1.5707963267948966 - a, a)


def _rowmath_body(hf, hc, sf, sc_, bias_f, bias_c, logc,
                  r_o, u_o, v_o, s_o, dh_o, ds_o):
    lg = logc[...]
    bf = bias_f[...]
    bc = bias_c[...]
    th = bf[:, 2:3] + bc[:, 3:4] - lg
    ts = bf[:, 4:5] + bc[:, 5:6] - lg
    te = bf[:, 0:1] + bc[:, 1:2] - lg
    le = (te - 1.0) ** 2
    r_o[...] = 0.5 * le + 0.25 * (th * th + ts * ts)
    u_o[...] = -0.5 * th
    v_o[...] = -0.5 * ts

    nmh = _stereo_nm(hf[...], hc[...], -0.5)
    t = jnp.clip(_SK * nmh, -1.0 + 1e-5, 1.0 - 1e-5)
    dist = jnp.log((1.0 + t) / (1.0 - t)) / _SK
    e = jnp.exp(dist)
    ch = 0.5 * (e + 1.0 / e)
    vh = ch * ch
    dh = vh / jnp.sqrt(jnp.sum(vh * vh))
    dh_o[...] = dh

    nms = _stereo_nm(sf[...], sc_[...], 0.5)
    dsts = 2.0 * _atan_pos(_SK * nms) / _SK
    vs = 0.5 * dsts * dsts
    ds = vs / jnp.sqrt(jnp.sum(vs * vs))
    ds_o[...] = ds

    s_o[...] = 0.25 * (dh * dh + ds * ds)


_rowmath = pl.pallas_call(
    _rowmath_body,
    out_shape=[jax.ShapeDtypeStruct((B_, 1), _f32)] * 6,
)


_TI = 256


def _outer_body(r, u, v, s, dh, ds, out):
    out[...] = (r[...] + s[...]) + u[...] * dh[...] + v[...] * ds[...]


_outer = pl.pallas_call(
    _outer_body,
    grid=(B_ // _TI,),
    in_specs=[
        pl.BlockSpec((_TI, 1), lambda i: (i, 0)),
        pl.BlockSpec((_TI, 1), lambda i: (i, 0)),
        pl.BlockSpec((_TI, 1), lambda i: (i, 0)),
        pl.BlockSpec((1, B_), lambda i: (0, 0)),
        pl.BlockSpec((1, B_), lambda i: (0, 0)),
        pl.BlockSpec((1, B_), lambda i: (0, 0)),
    ],
    out_specs=pl.BlockSpec((_TI, B_), lambda i: (i, 0)),
    out_shape=jax.ShapeDtypeStruct((B_, B_), _f32),
)


def kernel(focal_input, context_input, log_coocurrence_count,
           euc_fe, euc_ce, euc_fb, euc_cb,
           hyp_fe, hyp_ce, hyp_fb, hyp_cb,
           sph_fe, sph_ce, sph_fb, sph_cb):
    f = focal_input.astype(jnp.int32)
    c = context_input.astype(jnp.int32)
    bias_t = jnp.concatenate(
        [euc_fb, euc_cb, hyp_fb, hyp_cb, sph_fb, sph_cb, euc_fb, euc_cb],
        axis=1)
    (hf, hc, sf, sc_, bias_f, bias_c) = _make_sc_gather()(
        f, c, hyp_fe, hyp_ce, sph_fe, sph_ce, bias_t)
    r, u, v, s, dh, ds = _rowmath(hf, hc, sf, sc_, bias_f, bias_c,
                                  log_coocurrence_count)
    return _outer(r, u, v, s.reshape(1, B_), dh.reshape(1, B_),
                  ds.reshape(1, B_))


# speedup vs baseline: 1.4349x; 1.4243x over previous
import functools

import jax
import jax.numpy as jnp
from jax import lax
from jax.experimental import pallas as pl
from jax.experimental.pallas import tpu as pltpu
from jax.experimental.pallas import tpu_sc as plsc

VOCAB_ = 100000
DIM_ = 64
B_ = 4096

_NC = 2
_NS = 16
_NW = _NC * _NS
_BPW = B_ // _NW

_f32 = jnp.float32


def _sc_rows_body(focal, context, hfe, hce, sfe, sce,
                  hf_o, hc_o, sf_o, sc_o,
                  idxf_v, idxc_v, rhf, rhc, rsf, rsc, sem):
    wid = lax.axis_index("s") * _NC + lax.axis_index("c")
    base = wid * _BPW
    pltpu.sync_copy(focal.at[pl.ds(base, _BPW)], idxf_v)
    pltpu.sync_copy(context.at[pl.ds(base, _BPW)], idxc_v)

    def fetch(j, _):
        cf = idxf_v[pl.ds(j * 16, 16)]
        cc = idxc_v[pl.ds(j * 16, 16)]
        for t in range(16):
            i_f = cf[t]
            i_c = cc[t]
            r = j * 16 + t
            pltpu.async_copy(hfe.at[i_f], rhf.at[r], sem)
            pltpu.async_copy(hce.at[i_c], rhc.at[r], sem)
            pltpu.async_copy(sfe.at[i_f], rsf.at[r], sem)
            pltpu.async_copy(sce.at[i_c], rsc.at[r], sem)
        return 0
    lax.fori_loop(0, _BPW // 16, fetch, 0)

    pltpu.make_async_copy(hfe.at[pl.ds(0, _BPW)], rhf, sem).wait()
    pltpu.make_async_copy(hce.at[pl.ds(0, _BPW)], rhc, sem).wait()
    pltpu.make_async_copy(sfe.at[pl.ds(0, _BPW)], rsf, sem).wait()
    pltpu.make_async_copy(sce.at[pl.ds(0, _BPW)], rsc, sem).wait()

    s = pl.ds(base, _BPW)
    pltpu.sync_copy(rhf, hf_o.at[s])
    pltpu.sync_copy(rhc, hc_o.at[s])
    pltpu.sync_copy(rsf, sf_o.at[s])
    pltpu.sync_copy(rsc, sc_o.at[s])


@functools.lru_cache(maxsize=1)
def _make_sc_rows():
    return functools.partial(
        pl.kernel,
        mesh=plsc.VectorSubcoreMesh(core_axis_name="c", subcore_axis_name="s"),
        out_type=[jax.ShapeDtypeStruct((B_, DIM_), _f32)] * 4,
        scratch_types=[
            pltpu.VMEM((_BPW,), jnp.int32),
            pltpu.VMEM((_BPW,), jnp.int32),
            pltpu.VMEM((_BPW, DIM_), _f32),
            pltpu.VMEM((_BPW, DIM_), _f32),
            pltpu.VMEM((_BPW, DIM_), _f32),
            pltpu.VMEM((_BPW, DIM_), _f32),
            pltpu.SemaphoreType.DMA,
        ],
    )(_sc_rows_body)


def _sc_bias_body(focal, context, bfe8, bce8, bfh8, bch8, bfs8, bcs8,
                  bfe_o, bce_o, bfh_o, bch_o, bfs_o, bcs_o,
                  idxf_v, idxc_v, idxf8_v, idxc8_v,
                  b0, b1, b2, b3, b4, b5, sem):
    wid = lax.axis_index("s") * _NC + lax.axis_index("c")
    base = wid * _BPW
    pltpu.sync_copy(focal.at[pl.ds(base, _BPW)], idxf_v)
    pltpu.sync_copy(context.at[pl.ds(base, _BPW)], idxc_v)

    def shift_chunk(j, _):
        sl = pl.ds(j * 16, 16)
        idxf8_v[sl] = lax.shift_right_logical(idxf_v[sl], 3)
        idxc8_v[sl] = lax.shift_right_logical(idxc_v[sl], 3)
        return 0
    lax.fori_loop(0, _BPW // 16, shift_chunk, 0)

    cps = [
        pltpu.async_copy(bfe8.at[idxf8_v], b0, sem),
        pltpu.async_copy(bce8.at[idxc8_v], b1, sem),
        pltpu.async_copy(bfh8.at[idxf8_v], b2, sem),
        pltpu.async_copy(bch8.at[idxc8_v], b3, sem),
        pltpu.async_copy(bfs8.at[idxf8_v], b4, sem),
        pltpu.async_copy(bcs8.at[idxc8_v], b5, sem),
    ]
    for cp in cps:
        cp.wait()

    s = pl.ds(base, _BPW)
    pltpu.sync_copy(b0, bfe_o.at[s])
    pltpu.sync_copy(b1, bce_o.at[s])
    pltpu.sync_copy(b2, bfh_o.at[s])
    pltpu.sync_copy(b3, bch_o.at[s])
    pltpu.sync_copy(b4, bfs_o.at[s])
    pltpu.sync_copy(b5, bcs_o.at[s])


@functools.lru_cache(maxsize=1)
def _make_sc_bias():
    return functools.partial(
        pl.kernel,
        mesh=plsc.VectorSubcoreMesh(core_axis_name="c", subcore_axis_name="s"),
        compiler_params=pltpu.CompilerParams(use_tc_tiling_on_sc=False),
        out_type=[jax.ShapeDtypeStruct((B_, 8), _f32)] * 6,
        scratch_types=[
            pltpu.VMEM((_BPW,), jnp.int32),
            pltpu.VMEM((_BPW,), jnp.int32),
            pltpu.VMEM((_BPW,), jnp.int32),
            pltpu.VMEM((_BPW,), jnp.int32),
            pltpu.VMEM((_BPW, 8), _f32),
            pltpu.VMEM((_BPW, 8), _f32),
            pltpu.VMEM((_BPW, 8), _f32),
            pltpu.VMEM((_BPW, 8), _f32),
            pltpu.VMEM((_BPW, 8), _f32),
            pltpu.VMEM((_BPW, 8), _f32),
            pltpu.SemaphoreType.DMA,
        ],
    )(_sc_bias_body)


_SK = 0.7071067811865476


def _atan_pos(z):
    big = z > 1.0
    w = jnp.where(big, 1.0 / jnp.maximum(z, 1e-30), z)
    w = w / (1.0 + jnp.sqrt(1.0 + w * w))
    w = w / (1.0 + jnp.sqrt(1.0 + w * w))
    w2 = w * w
    p = w * (1.0 + w2 * (-1.0 / 3.0 + w2 * (0.2 + w2 * (-1.0 / 7.0
                                                        + w2 / 9.0))))
    a = 4.0 * p
    return jnp.where(big, 1.5707963267948966 - a, a)


def _stereo_nm(xr, yr, k):
    x = xr / jnp.sqrt(jnp.sum(xr * xr, axis=1, keepdims=True))
    y = yr / jnp.sqrt(jnp.sum(yr * yr, axis=1, keepdims=True))
    x2 = jnp.sum(x * x, axis=-1, keepdims=True)
    y2 = jnp.sum(y * y, axis=-1, keepdims=True)
    xy = jnp.sum(x * y, axis=-1, keepdims=True)
    num = (1.0 + k * x2) * y - (1.0 + 2.0 * k * xy - k * y2) * x
    den = 1.0 + 2.0 * k * xy + (k * k) * x2 * y2
    m = num / jnp.maximum(den, 1e-15)
    return jnp.sqrt(jnp.sum(m * m, axis=-1, keepdims=True))


def _bsel(rows8, mod):
    lane = lax.broadcasted_iota(jnp.int32, (1, 8), 1)
    return jnp.sum(jnp.where(lane == mod, rows8, 0.0), axis=1,
                   keepdims=True)


def _rowmath_body(hf, hc, sf, sc_, bfe8, bce8, bfh8, bch8, bfs8, bcs8,
                  fidx, cidx, logc, r_o, u_o, v_o, s_o, dh_o, ds_o):
    lg = logc[...]
    modf = lax.bitwise_and(fidx[...], 7)
    modc = lax.bitwise_and(cidx[...], 7)
    th = _bsel(bfh8[...], modf) + _bsel(bch8[...], modc) - lg
    ts = _bsel(bfs8[...], modf) + _bsel(bcs8[...], modc) - lg
    te = _bsel(bfe8[...], modf) + _bsel(bce8[...], modc) - lg
    le = (te - 1.0) ** 2
    r_o[...] = 0.5 * le + 0.25 * (th * th + ts * ts)
    u_o[...] = -0.5 * th
    v_o[...] = -0.5 * ts

    nmh = _stereo_nm(hf[...], hc[...], -0.5)
    t = jnp.clip(_SK * nmh, -1.0 + 1e-5, 1.0 - 1e-5)
    dist = jnp.log((1.0 + t) / (1.0 - t)) / _SK
    e = jnp.exp(dist)
    ch = 0.5 * (e + 1.0 / e)
    vh = ch * ch
    dh = vh / jnp.sqrt(jnp.sum(vh * vh))
    dh_o[...] = dh

    nms = _stereo_nm(sf[...], sc_[...], 0.5)
    dsts = 2.0 * _atan_pos(_SK * nms) / _SK
    vs = 0.5 * dsts * dsts
    ds = vs / jnp.sqrt(jnp.sum(vs * vs))
    ds_o[...] = ds

    s_o[...] = 0.25 * (dh * dh + ds * ds)


_rowmath = pl.pallas_call(
    _rowmath_body,
    out_shape=[jax.ShapeDtypeStruct((B_, 1), _f32)] * 6,
)


_TI = 256


def _outer_body(r, u, v, s, dh, ds, out):
    out[...] = (r[...] + s[...]) + u[...] * dh[...] + v[...] * ds[...]


_outer = pl.pallas_call(
    _outer_body,
    grid=(B_ // _TI,),
    in_specs=[
        pl.BlockSpec((_TI, 1), lambda i: (i, 0)),
        pl.BlockSpec((_TI, 1), lambda i: (i, 0)),
        pl.BlockSpec((_TI, 1), lambda i: (i, 0)),
        pl.BlockSpec((1, B_), lambda i: (0, 0)),
        pl.BlockSpec((1, B_), lambda i: (0, 0)),
        pl.BlockSpec((1, B_), lambda i: (0, 0)),
    ],
    out_specs=pl.BlockSpec((_TI, B_), lambda i: (i, 0)),
    out_shape=jax.ShapeDtypeStruct((B_, B_), _f32),
)


def kernel(focal_input, context_input, log_coocurrence_count,
           euc_fe, euc_ce, euc_fb, euc_cb,
           hyp_fe, hyp_ce, hyp_fb, hyp_cb,
           sph_fe, sph_ce, sph_fb, sph_cb):
    f = focal_input.astype(jnp.int32)
    c = context_input.astype(jnp.int32)
    hf, hc, sf, sc_ = _make_sc_rows()(f, c, hyp_fe, hyp_ce, sph_fe, sph_ce)
    v8 = (VOCAB_ // 8, 8)
    bfe8, bce8, bfh8, bch8, bfs8, bcs8 = _make_sc_bias()(
        f, c, euc_fb.reshape(v8), euc_cb.reshape(v8),
        hyp_fb.reshape(v8), hyp_cb.reshape(v8),
        sph_fb.reshape(v8), sph_cb.reshape(v8))
    r, u, v, s, dh, ds = _rowmath(
        hf, hc, sf, sc_, bfe8, bce8, bfh8, bch8, bfs8, bcs8,
        f.reshape(B_, 1), c.reshape(B_, 1), log_coocurrence_count)
    return _outer(r, u, v, s.reshape(1, B_), dh.reshape(1, B_),
                  ds.reshape(1, B_))


# speedup vs baseline: 1.4511x; 1.0113x over previous
import functools

import jax
import jax.numpy as jnp
from jax import lax
from jax.experimental import pallas as pl
from jax.experimental.pallas import tpu as pltpu
from jax.experimental.pallas import tpu_sc as plsc

VOCAB_ = 100000
DIM_ = 64
B_ = 4096

_NC = 2
_NS = 16
_NW = _NC * _NS
_BPW = B_ // _NW

_f32 = jnp.float32


def _sc_rows_body(focal, context, hfe, hce, sfe, sce,
                  hf_o, hc_o, sf_o, sc_o,
                  idxf_v, idxc_v, rhf, rhc, rsf, rsc, sem):
    wid = lax.axis_index("s") * _NC + lax.axis_index("c")
    base = wid * _BPW
    pltpu.sync_copy(focal.at[pl.ds(base, _BPW)], idxf_v)
    pltpu.sync_copy(context.at[pl.ds(base, _BPW)], idxc_v)

    def fetch(j, _):
        cf = idxf_v[pl.ds(j * 16, 16)]
        cc = idxc_v[pl.ds(j * 16, 16)]
        for t in range(16):
            i_f = cf[t]
            i_c = cc[t]
            r = j * 16 + t
            pltpu.async_copy(hfe.at[i_f], rhf.at[r], sem)
            pltpu.async_copy(hce.at[i_c], rhc.at[r], sem)
            pltpu.async_copy(sfe.at[i_f], rsf.at[r], sem)
            pltpu.async_copy(sce.at[i_c], rsc.at[r], sem)
        return 0
    lax.fori_loop(0, _BPW // 16, fetch, 0)

    pltpu.make_async_copy(hfe.at[pl.ds(0, _BPW)], rhf, sem).wait()
    pltpu.make_async_copy(hce.at[pl.ds(0, _BPW)], rhc, sem).wait()
    pltpu.make_async_copy(sfe.at[pl.ds(0, _BPW)], rsf, sem).wait()
    pltpu.make_async_copy(sce.at[pl.ds(0, _BPW)], rsc, sem).wait()

    s = pl.ds(base, _BPW)
    pltpu.sync_copy(rhf, hf_o.at[s])
    pltpu.sync_copy(rhc, hc_o.at[s])
    pltpu.sync_copy(rsf, sf_o.at[s])
    pltpu.sync_copy(rsc, sc_o.at[s])


@functools.lru_cache(maxsize=1)
def _make_sc_rows():
    return functools.partial(
        pl.kernel,
        mesh=plsc.VectorSubcoreMesh(core_axis_name="c", subcore_axis_name="s"),
        out_type=[jax.ShapeDtypeStruct((B_, DIM_), _f32)] * 4,
        scratch_types=[
            pltpu.VMEM((_BPW,), jnp.int32),
            pltpu.VMEM((_BPW,), jnp.int32),
            pltpu.VMEM((_BPW, DIM_), _f32),
            pltpu.VMEM((_BPW, DIM_), _f32),
            pltpu.VMEM((_BPW, DIM_), _f32),
            pltpu.VMEM((_BPW, DIM_), _f32),
            pltpu.SemaphoreType.DMA,
        ],
    )(_sc_rows_body)


def _sc_bias_body(focal, context, bfe8, bce8, bfh8, bch8, bfs8, bcs8,
                  bfe_o, bce_o, bfh_o, bch_o, bfs_o, bcs_o,
                  idxf_v, idxc_v, idxf8_v, idxc8_v,
                  b0, b1, b2, b3, b4, b5, sem):
    wid = lax.axis_index("s") * _NC + lax.axis_index("c")
    base = wid * _BPW
    pltpu.sync_copy(focal.at[pl.ds(base, _BPW)], idxf_v)
    pltpu.sync_copy(context.at[pl.ds(base, _BPW)], idxc_v)

    def shift_chunk(j, _):
        sl = pl.ds(j * 16, 16)
        idxf8_v[sl] = lax.shift_right_logical(idxf_v[sl], 3)
        idxc8_v[sl] = lax.shift_right_logical(idxc_v[sl], 3)
        return 0
    lax.fori_loop(0, _BPW // 16, shift_chunk, 0)

    cps = [
        pltpu.async_copy(bfe8.at[idxf8_v], b0, sem),
        pltpu.async_copy(bce8.at[idxc8_v], b1, sem),
        pltpu.async_copy(bfh8.at[idxf8_v], b2, sem),
        pltpu.async_copy(bch8.at[idxc8_v], b3, sem),
        pltpu.async_copy(bfs8.at[idxf8_v], b4, sem),
        pltpu.async_copy(bcs8.at[idxc8_v], b5, sem),
    ]
    for cp in cps:
        cp.wait()

    s = pl.ds(base, _BPW)
    pltpu.sync_copy(b0, bfe_o.at[s])
    pltpu.sync_copy(b1, bce_o.at[s])
    pltpu.sync_copy(b2, bfh_o.at[s])
    pltpu.sync_copy(b3, bch_o.at[s])
    pltpu.sync_copy(b4, bfs_o.at[s])
    pltpu.sync_copy(b5, bcs_o.at[s])


@functools.lru_cache(maxsize=1)
def _make_sc_bias():
    return functools.partial(
        pl.kernel,
        mesh=plsc.VectorSubcoreMesh(core_axis_name="c", subcore_axis_name="s"),
        compiler_params=pltpu.CompilerParams(use_tc_tiling_on_sc=False),
        out_type=[jax.ShapeDtypeStruct((B_, 8), _f32)] * 6,
        scratch_types=[
            pltpu.VMEM((_BPW,), jnp.int32),
            pltpu.VMEM((_BPW,), jnp.int32),
            pltpu.VMEM((_BPW,), jnp.int32),
            pltpu.VMEM((_BPW,), jnp.int32),
            pltpu.VMEM((_BPW, 8), _f32),
            pltpu.VMEM((_BPW, 8), _f32),
            pltpu.VMEM((_BPW, 8), _f32),
            pltpu.VMEM((_BPW, 8), _f32),
            pltpu.VMEM((_BPW, 8), _f32),
            pltpu.VMEM((_BPW, 8), _f32),
            pltpu.SemaphoreType.DMA,
        ],
    )(_sc_bias_body)


_SK = 0.7071067811865476


def _atan_pos(z):
    big = z > 1.0
    w = jnp.where(big, 1.0 / jnp.maximum(z, 1e-30), z)
    w = w / (1.0 + jnp.sqrt(1.0 + w * w))
    w = w / (1.0 + jnp.sqrt(1.0 + w * w))
    w2 = w * w
    p = w * (1.0 + w2 * (-1.0 / 3.0 + w2 * (0.2 + w2 * (-1.0 / 7.0
                                                        + w2 / 9.0))))
    a = 4.0 * p
    return jnp.where(big, 1.5707963267948966 - a, a)


def _stereo_nm(xr, yr, k):
    x = xr / jnp.sqrt(jnp.sum(xr * xr, axis=1, keepdims=True))
    y = yr / jnp.sqrt(jnp.sum(yr * yr, axis=1, keepdims=True))
    x2 = jnp.sum(x * x, axis=-1, keepdims=True)
    y2 = jnp.sum(y * y, axis=-1, keepdims=True)
    xy = jnp.sum(x * y, axis=-1, keepdims=True)
    num = (1.0 + k * x2) * y - (1.0 + 2.0 * k * xy - k * y2) * x
    den = 1.0 + 2.0 * k * xy + (k * k) * x2 * y2
    m = num / jnp.maximum(den, 1e-15)
    return jnp.sqrt(jnp.sum(m * m, axis=-1, keepdims=True))


def _bsel(rows8, mod):
    lane = lax.broadcasted_iota(jnp.int32, (1, 8), 1)
    return jnp.sum(jnp.where(lane == mod, rows8, 0.0), axis=1,
                   keepdims=True)


def _rowmath_body(hf, hc, sf, sc_, bfe8, bce8, bfh8, bch8, bfs8, bcs8,
                  fidx, cidx, logc, r_o, u_o, v_o, s_o, dh_o, ds_o):
    lg = logc[...]
    modf = lax.bitwise_and(fidx[...], 7)
    modc = lax.bitwise_and(cidx[...], 7)
    th = _bsel(bfh8[...], modf) + _bsel(bch8[...], modc) - lg
    ts = _bsel(bfs8[...], modf) + _bsel(bcs8[...], modc) - lg
    te = _bsel(bfe8[...], modf) + _bsel(bce8[...], modc) - lg
    le = (te - 1.0) ** 2
    r_o[...] = 0.5 * le + 0.25 * (th * th + ts * ts)
    u_o[...] = -0.5 * th
    v_o[...] = -0.5 * ts

    nmh = _stereo_nm(hf[...], hc[...], -0.5)
    t = jnp.clip(_SK * nmh, -1.0 + 1e-5, 1.0 - 1e-5)
    dist = jnp.log((1.0 + t) / (1.0 - t)) / _SK
    e = jnp.exp(dist)
    ch = 0.5 * (e + 1.0 / e)
    vh = ch * ch
    dh = vh / jnp.sqrt(jnp.sum(vh * vh))
    dh_o[...] = dh

    nms = _stereo_nm(sf[...], sc_[...], 0.5)
    dsts = 2.0 * _atan_pos(_SK * nms) / _SK
    vs = 0.5 * dsts * dsts
    ds = vs / jnp.sqrt(jnp.sum(vs * vs))
    ds_o[...] = ds

    s_o[...] = 0.25 * (dh * dh + ds * ds)


_rowmath = pl.pallas_call(
    _rowmath_body,
    out_shape=[jax.ShapeDtypeStruct((B_, 1), _f32)] * 6,
)


_TI = 512


def _outer_body(r, u, v, s, dh, ds, out):
    out[...] = (r[...] + s[...]) + u[...] * dh[...] + v[...] * ds[...]


_outer = pl.pallas_call(
    _outer_body,
    grid=(B_ // _TI,),
    in_specs=[
        pl.BlockSpec((_TI, 1), lambda i: (i, 0)),
        pl.BlockSpec((_TI, 1), lambda i: (i, 0)),
        pl.BlockSpec((_TI, 1), lambda i: (i, 0)),
        pl.BlockSpec((1, B_), lambda i: (0, 0)),
        pl.BlockSpec((1, B_), lambda i: (0, 0)),
        pl.BlockSpec((1, B_), lambda i: (0, 0)),
    ],
    out_specs=pl.BlockSpec((_TI, B_), lambda i: (i, 0)),
    out_shape=jax.ShapeDtypeStruct((B_, B_), _f32),
)


def kernel(focal_input, context_input, log_coocurrence_count,
           euc_fe, euc_ce, euc_fb, euc_cb,
           hyp_fe, hyp_ce, hyp_fb, hyp_cb,
           sph_fe, sph_ce, sph_fb, sph_cb):
    f = focal_input.astype(jnp.int32)
    c = context_input.astype(jnp.int32)
    hf, hc, sf, sc_ = _make_sc_rows()(f, c, hyp_fe, hyp_ce, sph_fe, sph_ce)
    v8 = (VOCAB_ // 8, 8)
    bfe8, bce8, bfh8, bch8, bfs8, bcs8 = _make_sc_bias()(
        f, c, euc_fb.reshape(v8), euc_cb.reshape(v8),
        hyp_fb.reshape(v8), hyp_cb.reshape(v8),
        sph_fb.reshape(v8), sph_cb.reshape(v8))
    r, u, v, s, dh, ds = _rowmath(
        hf, hc, sf, sc_, bfe8, bce8, bfh8, bch8, bfs8, bcs8,
        f.reshape(B_, 1), c.reshape(B_, 1), log_coocurrence_count)
    return _outer(r, u, v, s.reshape(1, B_), dh.reshape(1, B_),
                  ds.reshape(1, B_))
